# Initial kernel scaffold; baseline (speedup 1.0000x reference)
import jax
import jax.numpy as jnp
from jax.experimental import pallas as pl


def kernel(x, edge_indices, W):
    raise NotImplementedError("write your pallas kernel here")


# speedup vs baseline: 31.5132x; 31.5132x over previous
import functools

import jax
import jax.numpy as jnp
from jax import lax
from jax.experimental import pallas as pl
from jax.experimental.pallas import tpu as pltpu
from jax.experimental.pallas import tpu_sc as plsc

N = 10000
E = 320000
D = 128

NC, NS = 2, 16
NW = NC * NS

CHUNK = 128
CPT = 80
EPT = CPT * CHUNK
EPAD = NW * EPT

NPAD = 10112
PADN = NPAD - N
RPS = NPAD // NS
ZF, ZREM = 4, 120

GRP = EPT // 16

_sc_mesh = functools.partial(
    plsc.VectorSubcoreMesh, core_axis_name="c", subcore_axis_name="s",
    num_cores=NC, num_subcores=NS)

_sc_params = pltpu.CompilerParams(needs_layout_passes=False)


def _deg_body(src_hbm, out_hbm, idxv, hist):
    c = lax.axis_index("c")
    s = lax.axis_index("s")
    wid = s * NC + c
    pltpu.sync_copy(src_hbm.at[wid], idxv)

    z16 = jnp.zeros((16,), jnp.float32)

    def zb(i, _):
        hist[pl.ds(i * 16, 16)] = z16

    lax.fori_loop(0, NPAD // 16, zb, None, unroll=4)

    ones = jnp.ones((16,), jnp.float32)

    def hb(i, _):
        idx = idxv[pl.ds(i * 16, 16)]
        plsc.addupdate_scatter(hist, [idx], ones)

    lax.fori_loop(0, GRP, hb, None, unroll=4)
    pltpu.sync_copy(hist, out_hbm.at[wid])


_deg_call = pl.kernel(
    _deg_body,
    out_type=jax.ShapeDtypeStruct((NW, NPAD), jnp.float32),
    mesh=_sc_mesh(),
    scratch_types=[
        pltpu.VMEM((EPT,), jnp.int32),
        pltpu.VMEM((NPAD,), jnp.float32),
    ],
    compiler_params=_sc_params,
)


def _mm_body(x_ref, wt_ref, degt_ref, g_ref, dr_ref):
    deg = jnp.sum(degt_ref[...], axis=1, keepdims=True)
    dr = lax.rsqrt(deg + 1.0)
    dr_ref[...] = dr
    h = jnp.dot(x_ref[...], wt_ref[...],
                preferred_element_type=jnp.float32)
    g_ref[0:N, :] = dr[0:N] * h
    g_ref[N:NPAD, :] = jnp.zeros((PADN, D), jnp.float32)


_mm_call = pl.pallas_call(
    _mm_body,
    out_shape=(
        jax.ShapeDtypeStruct((NPAD, D), jnp.float32),
        jax.ShapeDtypeStruct((NPAD, 1), jnp.float32),
    ),
)


def _acc_body(src_hbm, dst_hbm, g_hbm, out_hbm,
              srcv, dstv, rows, acc, sem0, sem1):
    c = lax.axis_index("c")
    s = lax.axis_index("s")
    wid = s * NC + c
    pltpu.sync_copy(src_hbm.at[wid], srcv)
    pltpu.sync_copy(dst_hbm.at[wid], dstv)

    z16 = jnp.zeros((16,), jnp.float32)

    def zb(i, _):
        for j in range(8):
            rows[0, i, pl.ds(j * 16, 16)] = z16

    lax.fori_loop(0, CHUNK, zb, None, unroll=2)

    base = s * RPS
    for j in range(ZF):
        pltpu.sync_copy(rows.at[0], acc.at[pl.ds(base + j * CHUNK, CHUNK)])
    pltpu.sync_copy(rows.at[0, pl.ds(0, ZREM)],
                    acc.at[pl.ds(base + ZF * CHUNK, ZREM)])
    plsc.subcore_barrier()

    def gather(chunk):
        return pltpu.make_async_copy(
            g_hbm.at[dstv.at[chunk]], rows.at[0], sem0)

    def lb(chunk, _):
        gather(chunk).wait()
        pltpu.sync_copy(rows.at[0], acc.at[srcv.at[chunk]], add=True)

        @pl.when(chunk < CPT - 1)
        def _start():
            gather(chunk + 1).start()

    gather(0).start()
    lax.fori_loop(0, CPT, lb, None)

    plsc.subcore_barrier()
    pltpu.sync_copy(acc.at[pl.ds(base, RPS)],
                    out_hbm.at[c, pl.ds(base, RPS)])


_acc_call = pl.kernel(
    _acc_body,
    out_type=jax.ShapeDtypeStruct((NC, NPAD, D), jnp.float32),
    mesh=_sc_mesh(),
    scratch_types=[
        pltpu.VMEM((CPT, CHUNK), jnp.int32),
        pltpu.VMEM((CPT, CHUNK), jnp.int32),
        pltpu.VMEM((1, CHUNK, D), jnp.float32),
        pltpu.VMEM_SHARED((NPAD, D), jnp.float32),
        pltpu.SemaphoreType.DMA,
        pltpu.SemaphoreType.DMA,
    ],
    compiler_params=_sc_params,
)


def _out_body(acc_ref, g_ref, dr_ref, out_ref):
    tot = acc_ref[0, 0:N, :] + acc_ref[1, 0:N, :] + g_ref[0:N, :]
    out_ref[...] = jnp.maximum(dr_ref[0:N] * tot, 0.0)


_out_call = pl.pallas_call(
    _out_body,
    out_shape=jax.ShapeDtypeStruct((N, D), jnp.float32),
)


def kernel(x, edge_indices, W):
    ei = edge_indices.astype(jnp.int32)
    pad = (jnp.arange(EPAD - E, dtype=jnp.int32) % PADN) + N
    src = jnp.concatenate([ei[0], pad])
    dst = jnp.concatenate([ei[1], pad])

    deg_parts = _deg_call(src.reshape(NW, EPT))
    degt = deg_parts.T
    g, dr = _mm_call(x, W.T, degt)
    acc = _acc_call(src.reshape(NW, CPT, CHUNK),
                    dst.reshape(NW, CPT, CHUNK), g)
    return _out_call(acc, g, dr)


# speedup vs baseline: 41.2558x; 1.3092x over previous
import functools

import jax
import jax.numpy as jnp
from jax import lax
from jax.experimental import pallas as pl
from jax.experimental.pallas import tpu as pltpu
from jax.experimental.pallas import tpu_sc as plsc

N = 10000
E = 320000
D = 128

NC, NS = 2, 16
NW = NC * NS

CHUNK = 128
CPT = 80
EPT = CPT * CHUNK
EPAD = NW * EPT

NPAD = 10112
PADN = NPAD - N
RPS = NPAD // NS
ZF, ZREM = 4, 120

GRP = EPT // 16

_sc_mesh = functools.partial(
    plsc.VectorSubcoreMesh, core_axis_name="c", subcore_axis_name="s",
    num_cores=NC, num_subcores=NS)

_sc_params = pltpu.CompilerParams(needs_layout_passes=False)


def _deg_body(src_hbm, out_hbm, idxv, hist):
    c = lax.axis_index("c")
    s = lax.axis_index("s")
    wid = s * NC + c
    pltpu.sync_copy(src_hbm.at[wid], idxv)

    z16 = jnp.zeros((16,), jnp.float32)

    def zb(i, _):
        hist[pl.ds(i * 16, 16)] = z16

    lax.fori_loop(0, NPAD // 16, zb, None, unroll=4)

    ones = jnp.ones((16,), jnp.float32)

    def hb(i, _):
        idx = idxv[pl.ds(i * 16, 16)]
        plsc.addupdate_scatter(hist, [idx], ones)

    lax.fori_loop(0, GRP, hb, None, unroll=4)
    pltpu.sync_copy(hist, out_hbm.at[wid])


_deg_call = pl.kernel(
    _deg_body,
    out_type=jax.ShapeDtypeStruct((NW, NPAD), jnp.float32),
    mesh=_sc_mesh(),
    scratch_types=[
        pltpu.VMEM((EPT,), jnp.int32),
        pltpu.VMEM((NPAD,), jnp.float32),
    ],
    compiler_params=_sc_params,
)


def _mm_body(x_ref, wt_ref, degt_ref, g_ref, dr_ref):
    deg = jnp.sum(degt_ref[...], axis=1, keepdims=True)
    dr = lax.rsqrt(deg + 1.0)
    dr_ref[...] = dr
    h = jnp.dot(x_ref[...], wt_ref[...],
                preferred_element_type=jnp.float32)
    g_ref[0:N, :] = dr[0:N] * h
    g_ref[N:NPAD, :] = jnp.zeros((PADN, D), jnp.float32)


_mm_call = pl.pallas_call(
    _mm_body,
    out_shape=(
        jax.ShapeDtypeStruct((NPAD, D), jnp.float32),
        jax.ShapeDtypeStruct((NPAD, 1), jnp.float32),
    ),
)


def _acc_body(e_hbm, g_hbm, out_hbm,
              idxb, rows, acc, semi0, semi1, semg0, semg1):
    c = lax.axis_index("c")
    s = lax.axis_index("s")
    wid = s * NC + c

    semi = (semi0, semi1)
    semg = (semg0, semg1)

    z16 = jnp.zeros((16,), jnp.float32)

    def zb(i, _):
        for j in range(8):
            rows[0, i, pl.ds(j * 16, 16)] = z16

    lax.fori_loop(0, CHUNK, zb, None, unroll=2)

    base = s * RPS
    for j in range(ZF):
        pltpu.sync_copy(rows.at[0], acc.at[pl.ds(base + j * CHUNK, CHUNK)])
    pltpu.sync_copy(rows.at[0, pl.ds(0, ZREM)],
                    acc.at[pl.ds(base + ZF * CHUNK, ZREM)])
    plsc.subcore_barrier()

    def idxload(chunk, b):
        return pltpu.make_async_copy(e_hbm.at[wid, chunk], idxb.at[b],
                                     semi[b])

    def gather(b):
        return pltpu.make_async_copy(g_hbm.at[idxb.at[b, 0]], rows.at[b],
                                     semg[b])

    idxload(0, 0).start()
    idxload(1, 1).start()
    idxload(0, 0).wait()
    gather(0).start()

    def lb(jj, _):
        for b in range(2):
            chunk = jj * 2 + b
            nb = 1 - b
            if b == 0:
                idxload(chunk + 1, nb).wait()
                gather(nb).start()
            else:
                @pl.when(jj < CPT // 2 - 1)
                def _next():
                    idxload(chunk + 1, nb).wait()
                    gather(nb).start()
            gather(b).wait()
            pltpu.sync_copy(rows.at[b], acc.at[idxb.at[b, 1]], add=True)

            @pl.when(chunk + 2 < CPT)
            def _pref():
                idxload(chunk + 2, b).start()

    lax.fori_loop(0, CPT // 2, lb, None)

    plsc.subcore_barrier()
    pltpu.sync_copy(acc.at[pl.ds(base, RPS)],
                    out_hbm.at[c, pl.ds(base, RPS)])


_acc_call = pl.kernel(
    _acc_body,
    out_type=jax.ShapeDtypeStruct((NC, NPAD, D), jnp.float32),
    mesh=_sc_mesh(),
    scratch_types=[
        pltpu.VMEM((2, 2, CHUNK), jnp.int32),
        pltpu.VMEM((2, CHUNK, D), jnp.float32),
        pltpu.VMEM_SHARED((NPAD, D), jnp.float32),
        pltpu.SemaphoreType.DMA,
        pltpu.SemaphoreType.DMA,
        pltpu.SemaphoreType.DMA,
        pltpu.SemaphoreType.DMA,
    ],
    compiler_params=_sc_params,
)


def _out_body(acc_ref, g_ref, dr_ref, out_ref):
    tot = acc_ref[0, 0:N, :] + acc_ref[1, 0:N, :] + g_ref[0:N, :]
    out_ref[...] = jnp.maximum(dr_ref[0:N] * tot, 0.0)


_out_call = pl.pallas_call(
    _out_body,
    out_shape=jax.ShapeDtypeStruct((N, D), jnp.float32),
)


def kernel(x, edge_indices, W):
    ei = edge_indices.astype(jnp.int32)
    pad = (jnp.arange(EPAD - E, dtype=jnp.int32) % PADN) + N
    src = jnp.concatenate([ei[0], pad])
    dst = jnp.concatenate([ei[1], pad])

    deg_parts = _deg_call(src.reshape(NW, EPT))
    degt = deg_parts.T
    g, dr = _mm_call(x, W.T, degt)
    e4 = jnp.concatenate([dst.reshape(NW, CPT, 1, CHUNK),
                          src.reshape(NW, CPT, 1, CHUNK)], axis=2)
    acc = _acc_call(e4, g)
    return _out_call(acc, g, dr)


# speedup vs baseline: 45.0322x; 1.0915x over previous
import functools

import jax
import jax.numpy as jnp
from jax import lax
from jax.experimental import pallas as pl
from jax.experimental.pallas import tpu as pltpu
from jax.experimental.pallas import tpu_sc as plsc

N = 10000
E = 320000
D = 128

NC, NS = 2, 16
NW = NC * NS

CHUNK = 128
CPT = 80
EPT = CPT * CHUNK
EPAD = NW * EPT

NPAD = 10112
PADN = NPAD - N
RPS = NPAD // NS
ZF, ZREM = 4, 120

GRP = EPT // 16

_sc_mesh = functools.partial(
    plsc.VectorSubcoreMesh, core_axis_name="c", subcore_axis_name="s",
    num_cores=NC, num_subcores=NS)

_sc_params = pltpu.CompilerParams(needs_layout_passes=False)


def _deg_body(src_hbm, out_hbm, idxv, hist):
    c = lax.axis_index("c")
    s = lax.axis_index("s")
    wid = s * NC + c
    pltpu.sync_copy(src_hbm.at[wid], idxv)

    z16 = jnp.zeros((16,), jnp.float32)

    def zb(i, _):
        hist[pl.ds(i * 16, 16)] = z16

    lax.fori_loop(0, NPAD // 16, zb, None, unroll=4)

    ones = jnp.ones((16,), jnp.float32)

    def hb(i, _):
        idx = idxv[pl.ds(i * 16, 16)]
        plsc.addupdate_scatter(hist, [idx], ones)

    lax.fori_loop(0, GRP, hb, None, unroll=4)
    pltpu.sync_copy(hist, out_hbm.at[wid])


_deg_call = pl.kernel(
    _deg_body,
    out_type=jax.ShapeDtypeStruct((NW, NPAD), jnp.float32),
    mesh=_sc_mesh(),
    scratch_types=[
        pltpu.VMEM((EPT,), jnp.int32),
        pltpu.VMEM((NPAD,), jnp.float32),
    ],
    compiler_params=_sc_params,
)


def _mm_body(x_ref, wt_ref, degt_ref, g_ref, dr_ref):
    deg = jnp.sum(degt_ref[...], axis=1, keepdims=True)
    dr = lax.rsqrt(deg + 1.0)
    dr_ref[...] = dr
    h = jnp.dot(x_ref[...], wt_ref[...],
                preferred_element_type=jnp.float32)
    g_ref[0:N, :] = dr[0:N] * h
    g_ref[N:NPAD, :] = jnp.zeros((PADN, D), jnp.float32)


_mm_call = pl.pallas_call(
    _mm_body,
    out_shape=(
        jax.ShapeDtypeStruct((NPAD, D), jnp.float32),
        jax.ShapeDtypeStruct((NPAD, 1), jnp.float32),
    ),
)


def _acc_body(e_hbm, g_hbm, out_hbm, idxb, rows, acc,
              semi0, semi1, semi2, semi3, semg0, semg1, sems0, sems1):
    c = lax.axis_index("c")
    s = lax.axis_index("s")
    wid = s * NC + c

    semi = (semi0, semi1, semi2, semi3)
    semg = (semg0, semg1)
    semsc = (sems0, sems1)

    z16 = jnp.zeros((16,), jnp.float32)

    def zb(i, _):
        for j in range(8):
            rows[0, i, pl.ds(j * 16, 16)] = z16

    lax.fori_loop(0, CHUNK, zb, None, unroll=2)

    base = s * RPS
    for j in range(ZF):
        pltpu.sync_copy(rows.at[0], acc.at[pl.ds(base + j * CHUNK, CHUNK)])
    pltpu.sync_copy(rows.at[0, pl.ds(0, ZREM)],
                    acc.at[pl.ds(base + ZF * CHUNK, ZREM)])
    plsc.subcore_barrier()

    def idxload(chunk, i):
        return pltpu.make_async_copy(e_hbm.at[wid, chunk], idxb.at[i],
                                     semi[i])

    def gather(i4, b):
        return pltpu.make_async_copy(
            g_hbm.at[idxb.at[i4, 0]], rows.at[b], semg[b])

    def scatter_start(i4, b):
        pltpu.async_copy(rows.at[b], acc.at[idxb.at[i4, 1]], semsc[b],
                         add=True)

    def scatter_wait(i4, b):
        pltpu.make_async_copy(rows.at[b], acc.at[idxb.at[i4, 1]],
                              semsc[b]).wait()

    for k in range(3):
        idxload(k, k).start()
    idxload(0, 0).wait()
    gather(0, 0).start()

    def lb(jj2, _):
        for p in range(2):
            for b in range(2):
                i4 = 2 * p + b
                nx = (i4 + 1) % 4
                pv = (i4 + 3) % 4
                chunk = jj2 * 4 + i4
                first = i4 == 0
                last = i4 == 3

                if not last:
                    idxload(chunk + 1, nx).wait()
                else:
                    @pl.when(chunk + 1 < CPT)
                    def _wi():
                        idxload(chunk + 1, nx).wait()

                if first:
                    @pl.when(chunk > 0)
                    def _ws():
                        scatter_wait(pv, 1)
                else:
                    scatter_wait(pv, 1 - b)

                if not last:
                    gather(nx, 1 - b).start()
                else:
                    @pl.when(chunk + 1 < CPT)
                    def _ng():
                        gather(nx, 1 - b).start()

                @pl.when(chunk + 3 < CPT)
                def _pi():
                    idxload(chunk + 3, pv).start()

                gather(i4, b).wait()
                scatter_start(i4, b)

    lax.fori_loop(0, CPT // 4, lb, None)
    scatter_wait((CPT - 1) % 4, (CPT - 1) % 2)

    plsc.subcore_barrier()
    pltpu.sync_copy(acc.at[pl.ds(base, RPS)],
                    out_hbm.at[c, pl.ds(base, RPS)])


_acc_call = pl.kernel(
    _acc_body,
    out_type=jax.ShapeDtypeStruct((NC, NPAD, D), jnp.float32),
    mesh=_sc_mesh(),
    scratch_types=[
        pltpu.VMEM((4, 2, CHUNK), jnp.int32),
        pltpu.VMEM((2, CHUNK, D), jnp.float32),
        pltpu.VMEM_SHARED((NPAD, D), jnp.float32),
    ] + [pltpu.SemaphoreType.DMA] * 8,
    compiler_params=_sc_params,
)


def _out_body(acc_ref, g_ref, dr_ref, out_ref):
    tot = acc_ref[0, 0:N, :] + acc_ref[1, 0:N, :] + g_ref[0:N, :]
    out_ref[...] = jnp.maximum(dr_ref[0:N] * tot, 0.0)


_out_call = pl.pallas_call(
    _out_body,
    out_shape=jax.ShapeDtypeStruct((N, D), jnp.float32),
)


def kernel(x, edge_indices, W):
    ei = edge_indices.astype(jnp.int32)
    pad = (jnp.arange(EPAD - E, dtype=jnp.int32) % PADN) + N
    src = jnp.concatenate([ei[0], pad])
    dst = jnp.concatenate([ei[1], pad])

    deg_parts = _deg_call(src.reshape(NW, EPT))
    degt = deg_parts.T
    g, dr = _mm_call(x, W.T, degt)
    e4 = jnp.concatenate([dst.reshape(NW, CPT, 1, CHUNK),
                          src.reshape(NW, CPT, 1, CHUNK)], axis=2)
    acc = _acc_call(e4, g)
    return _out_call(acc, g, dr)


# speedup vs baseline: 45.8685x; 1.0186x over previous
import functools

import jax
import jax.numpy as jnp
from jax import lax
from jax.experimental import pallas as pl
from jax.experimental.pallas import tpu as pltpu
from jax.experimental.pallas import tpu_sc as plsc

N = 10000
E = 320000
D = 128

NC, NS = 2, 16
NW = NC * NS

CHUNK = 128
CPT = 80
EPT = CPT * CHUNK
EPAD = NW * EPT

NPAD = 10112
RPS = NPAD // NS
ZF, ZREM = 4, 120

GRP = EPT // 16

_sc_mesh = functools.partial(
    plsc.VectorSubcoreMesh, core_axis_name="c", subcore_axis_name="s",
    num_cores=NC, num_subcores=NS)

_sc_params = pltpu.CompilerParams(needs_layout_passes=False)


def _deg_body(src_hbm, out_hbm, idxv, hist):
    c = lax.axis_index("c")
    s = lax.axis_index("s")
    wid = s * NC + c
    pltpu.sync_copy(src_hbm.at[wid], idxv)

    z16 = jnp.zeros((16,), jnp.float32)

    def zb(i, _):
        hist[pl.ds(i * 16, 16)] = z16

    lax.fori_loop(0, NPAD // 16, zb, None, unroll=4)

    ones = jnp.ones((16,), jnp.float32)

    def hb(i, _):
        idx = idxv[pl.ds(i * 16, 16)]
        plsc.addupdate_scatter(hist, [idx], ones)

    lax.fori_loop(0, GRP, hb, None, unroll=4)
    pltpu.sync_copy(hist, out_hbm.at[wid])


_deg_call = pl.kernel(
    _deg_body,
    out_type=jax.ShapeDtypeStruct((NW, NPAD), jnp.float32),
    mesh=_sc_mesh(),
    scratch_types=[
        pltpu.VMEM((EPT,), jnp.int32),
        pltpu.VMEM((NPAD,), jnp.float32),
    ],
    compiler_params=_sc_params,
)


def _mm_body(x_ref, w_ref, degp_ref, g_ref, dr_ref):
    onesw = jnp.ones((NW, 1), jnp.float32)
    deg = lax.dot_general(degp_ref[...], onesw, (((0,), (0,)), ((), ())),
                          preferred_element_type=jnp.float32)
    dr = lax.rsqrt(deg + 1.0)
    dr_ref[...] = dr
    h = lax.dot_general(x_ref[...], w_ref[...], (((1,), (1,)), ((), ())),
                        preferred_element_type=jnp.float32)
    g_ref[0:N, :] = dr[0:N] * h
    g_ref[N:NPAD, :] = jnp.zeros((NPAD - N, D), jnp.float32)


_mm_call = pl.pallas_call(
    _mm_body,
    out_shape=(
        jax.ShapeDtypeStruct((NPAD, D), jnp.float32),
        jax.ShapeDtypeStruct((NPAD, 1), jnp.float32),
    ),
)


def _acc_body(src_hbm, dst_hbm, g_hbm, out_hbm, idxb, rows, acc,
              semi0, semi1, semi2, semi3, semg0, semg1, sems0, sems1):
    c = lax.axis_index("c")
    s = lax.axis_index("s")
    wid = s * NC + c

    semi = (semi0, semi1, semi2, semi3)
    semg = (semg0, semg1)
    semsc = (sems0, sems1)

    z16 = jnp.zeros((16,), jnp.float32)

    def zb(i, _):
        for j in range(8):
            rows[0, i, pl.ds(j * 16, 16)] = z16

    lax.fori_loop(0, CHUNK, zb, None, unroll=2)

    base = s * RPS
    for j in range(ZF):
        pltpu.sync_copy(rows.at[0], acc.at[pl.ds(base + j * CHUNK, CHUNK)])
    pltpu.sync_copy(rows.at[0, pl.ds(0, ZREM)],
                    acc.at[pl.ds(base + ZF * CHUNK, ZREM)])
    plsc.subcore_barrier()

    def idx_copies(chunk, i):
        return [pltpu.make_async_copy(hbm.at[wid, chunk], idxb.at[i, d],
                                      semi[i])
                for d, hbm in ((0, dst_hbm), (1, src_hbm))]

    def istart(chunk, i):
        for cp in idx_copies(chunk, i):
            cp.start()

    def iwait(chunk, i):
        for cp in idx_copies(chunk, i):
            cp.wait()

    def gather(i4, b):
        return pltpu.make_async_copy(
            g_hbm.at[idxb.at[i4, 0]], rows.at[b], semg[b])

    def scatter_start(i4, b):
        pltpu.async_copy(rows.at[b], acc.at[idxb.at[i4, 1]], semsc[b],
                         add=True)

    def scatter_wait(i4, b):
        pltpu.make_async_copy(rows.at[b], acc.at[idxb.at[i4, 1]],
                              semsc[b]).wait()

    for k in range(3):
        istart(k, k)
    iwait(0, 0)
    gather(0, 0).start()

    def lb(jj2, _):
        for p in range(2):
            for b in range(2):
                i4 = 2 * p + b
                nx = (i4 + 1) % 4
                pv = (i4 + 3) % 4
                chunk = jj2 * 4 + i4
                last = i4 == 3

                if not last:
                    iwait(chunk + 1, nx)
                else:
                    @pl.when(chunk + 1 < CPT)
                    def _wi():
                        iwait(chunk + 1, nx)

                if i4 == 0:
                    @pl.when(chunk > 0)
                    def _ws():
                        scatter_wait(pv, 1)
                else:
                    scatter_wait(pv, 1 - b)

                if not last:
                    gather(nx, 1 - b).start()
                else:
                    @pl.when(chunk + 1 < CPT)
                    def _ng():
                        gather(nx, 1 - b).start()

                @pl.when(chunk + 3 < CPT)
                def _pi():
                    istart(chunk + 3, pv)

                gather(i4, b).wait()
                scatter_start(i4, b)

    lax.fori_loop(0, CPT // 4, lb, None)
    scatter_wait((CPT - 1) % 4, (CPT - 1) % 2)

    plsc.subcore_barrier()
    pltpu.sync_copy(acc.at[pl.ds(base, RPS)],
                    out_hbm.at[c, pl.ds(base, RPS)])


_acc_call = pl.kernel(
    _acc_body,
    out_type=jax.ShapeDtypeStruct((NC, NPAD, D), jnp.float32),
    mesh=_sc_mesh(),
    scratch_types=[
        pltpu.VMEM((4, 2, CHUNK), jnp.int32),
        pltpu.VMEM((2, CHUNK, D), jnp.float32),
        pltpu.VMEM_SHARED((NPAD, D), jnp.float32),
    ] + [pltpu.SemaphoreType.DMA] * 8,
    compiler_params=_sc_params,
)


def _out_body(acc_ref, g_ref, dr_ref, out_ref):
    tot = acc_ref[0, 0:N, :] + acc_ref[1, 0:N, :] + g_ref[0:N, :]
    out_ref[...] = jnp.maximum(dr_ref[0:N] * tot, 0.0)


_out_call = pl.pallas_call(
    _out_body,
    out_shape=jax.ShapeDtypeStruct((N, D), jnp.float32),
)


def kernel(x, edge_indices, W):
    ei = edge_indices.astype(jnp.int32)
    pad = (jnp.arange(EPAD - E, dtype=jnp.int32) % (NPAD - N)) + N
    src3 = jnp.concatenate([ei[0], pad]).reshape(NW, CPT, CHUNK)
    dst3 = jnp.concatenate([ei[1], pad]).reshape(NW, CPT, CHUNK)

    deg_parts = _deg_call(src3.reshape(NW, EPT))
    g, dr = _mm_call(x, W, deg_parts)
    acc = _acc_call(src3, dst3, g)
    return _out_call(acc, g, dr)


# speedup vs baseline: 47.0122x; 1.0249x over previous
import functools

import jax
import jax.numpy as jnp
from jax import lax
from jax.experimental import pallas as pl
from jax.experimental.pallas import tpu as pltpu
from jax.experimental.pallas import tpu_sc as plsc

N = 10000
E = 320000
D = 128

NC, NS = 2, 16
NW = NC * NS

CHUNK = 128
CPT = 80
EPT = CPT * CHUNK
EPAD = NW * EPT

NPAD = 10112
RPS = NPAD // NS
ZF, ZREM = 4, 120

GRP = EPT // 16

_sc_mesh = functools.partial(
    plsc.VectorSubcoreMesh, core_axis_name="c", subcore_axis_name="s",
    num_cores=NC, num_subcores=NS)

_sc_params = pltpu.CompilerParams(needs_layout_passes=False)


EPW = E // NW


def _deg_body(ei_hbm, out_hbm, idxv, hist):
    c = lax.axis_index("c")
    s = lax.axis_index("s")
    wid = s * NC + c
    astart = pl.multiple_of(
        jnp.minimum(wid * EPW - (16 * wid) % 128, E - EPT), 128)
    off = wid * EPW - astart
    pltpu.sync_copy(ei_hbm.at[:, pl.ds(astart, EPT)], idxv)

    z16 = jnp.zeros((16,), jnp.float32)

    def zb(i, _):
        hist[pl.ds(i * 16, 16)] = z16

    lax.fori_loop(0, NPAD // 16, zb, None, unroll=4)

    ones = jnp.ones((16,), jnp.float32)
    lanes = lax.iota(jnp.int32, 16)

    def hb(i, _):
        local = lanes + i * 16
        mask = (local >= off) & (local < off + EPW)
        idx = idxv[0, pl.ds(i * 16, 16)]
        plsc.addupdate_scatter(hist, [idx], ones, mask=mask)

    lax.fori_loop(0, GRP, hb, None, unroll=4)
    pltpu.sync_copy(hist, out_hbm.at[wid])


_deg_call = pl.kernel(
    _deg_body,
    out_type=jax.ShapeDtypeStruct((NW, NPAD), jnp.float32),
    mesh=_sc_mesh(),
    scratch_types=[
        pltpu.VMEM((2, EPT), jnp.int32),
        pltpu.VMEM((NPAD,), jnp.float32),
    ],
    compiler_params=_sc_params,
)


def _mm_body(x_ref, w_ref, degp_ref, g_ref, dr_ref):
    onesw = jnp.ones((NW, 1), jnp.float32)
    deg = lax.dot_general(degp_ref[...], onesw, (((0,), (0,)), ((), ())),
                          preferred_element_type=jnp.float32)
    dr = lax.rsqrt(deg + 1.0)
    dr_ref[...] = dr
    h = lax.dot_general(x_ref[...], w_ref[...], (((1,), (1,)), ((), ())),
                        preferred_element_type=jnp.float32)
    g_ref[0:N, :] = dr[0:N] * h
    g_ref[N:NPAD, :] = jnp.zeros((NPAD - N, D), jnp.float32)


_mm_call = pl.pallas_call(
    _mm_body,
    out_shape=(
        jax.ShapeDtypeStruct((NPAD, D), jnp.float32),
        jax.ShapeDtypeStruct((NPAD, 1), jnp.float32),
    ),
)


def _acc_body(src_hbm, dst_hbm, g_hbm, out_hbm, idxb, rows, acc,
              semi0, semi1, semi2, semi3, semg0, semg1, sems0, sems1):
    c = lax.axis_index("c")
    s = lax.axis_index("s")
    wid = s * NC + c

    semi = (semi0, semi1, semi2, semi3)
    semg = (semg0, semg1)
    semsc = (sems0, sems1)

    base = s * RPS

    @pl.when(c == 0)
    def _seed_g():
        for j in range(ZF):
            pltpu.sync_copy(g_hbm.at[pl.ds(base + j * CHUNK, CHUNK)],
                            acc.at[pl.ds(base + j * CHUNK, CHUNK)])
        pltpu.sync_copy(g_hbm.at[pl.ds(base + ZF * CHUNK, ZREM)],
                        acc.at[pl.ds(base + ZF * CHUNK, ZREM)])

    @pl.when(c == 1)
    def _seed_zero():
        for j in range(5):
            pltpu.sync_copy(g_hbm.at[pl.ds(N, 112)],
                            acc.at[pl.ds(base + j * 112, 112)])
        pltpu.sync_copy(g_hbm.at[pl.ds(N, 72)],
                        acc.at[pl.ds(base + 560, 72)])

    plsc.subcore_barrier()

    def idx_copies(chunk, i):
        return [pltpu.make_async_copy(hbm.at[wid, chunk], idxb.at[i, d],
                                      semi[i])
                for d, hbm in ((0, dst_hbm), (1, src_hbm))]

    def istart(chunk, i):
        for cp in idx_copies(chunk, i):
            cp.start()

    def iwait(chunk, i):
        for cp in idx_copies(chunk, i):
            cp.wait()

    def gather(i4, b):
        return pltpu.make_async_copy(
            g_hbm.at[idxb.at[i4, 0]], rows.at[b], semg[b])

    def scatter_start(i4, b):
        pltpu.async_copy(rows.at[b], acc.at[idxb.at[i4, 1]], semsc[b],
                         add=True)

    def scatter_wait(i4, b):
        pltpu.make_async_copy(rows.at[b], acc.at[idxb.at[i4, 1]],
                              semsc[b]).wait()

    for k in range(3):
        istart(k, k)
    iwait(0, 0)
    gather(0, 0).start()

    def lb(jj2, _):
        for p in range(2):
            for b in range(2):
                i4 = 2 * p + b
                nx = (i4 + 1) % 4
                pv = (i4 + 3) % 4
                chunk = jj2 * 4 + i4
                last = i4 == 3

                if not last:
                    iwait(chunk + 1, nx)
                else:
                    @pl.when(chunk + 1 < CPT)
                    def _wi():
                        iwait(chunk + 1, nx)

                if i4 == 0:
                    @pl.when(chunk > 0)
                    def _ws():
                        scatter_wait(pv, 1)
                else:
                    scatter_wait(pv, 1 - b)

                if not last:
                    gather(nx, 1 - b).start()
                else:
                    @pl.when(chunk + 1 < CPT)
                    def _ng():
                        gather(nx, 1 - b).start()

                @pl.when(chunk + 3 < CPT)
                def _pi():
                    istart(chunk + 3, pv)

                gather(i4, b).wait()
                scatter_start(i4, b)

    lax.fori_loop(0, CPT // 4, lb, None)
    scatter_wait((CPT - 1) % 4, (CPT - 1) % 2)

    plsc.subcore_barrier()
    pltpu.sync_copy(acc.at[pl.ds(base, RPS)],
                    out_hbm.at[c, pl.ds(base, RPS)])


_acc_call = pl.kernel(
    _acc_body,
    out_type=jax.ShapeDtypeStruct((NC, NPAD, D), jnp.float32),
    mesh=_sc_mesh(),
    scratch_types=[
        pltpu.VMEM((4, 2, CHUNK), jnp.int32),
        pltpu.VMEM((2, CHUNK, D), jnp.float32),
        pltpu.VMEM_SHARED((NPAD, D), jnp.float32),
    ] + [pltpu.SemaphoreType.DMA] * 8,
    compiler_params=_sc_params,
)


def _out_body(acc_ref, dr_ref, out_ref):
    tot = acc_ref[0, 0:N, :] + acc_ref[1, 0:N, :]
    out_ref[...] = jnp.maximum(dr_ref[0:N] * tot, 0.0)


_out_call = pl.pallas_call(
    _out_body,
    out_shape=jax.ShapeDtypeStruct((N, D), jnp.float32),
)


def kernel(x, edge_indices, W):
    ei = edge_indices.astype(jnp.int32)
    pad = (jnp.arange(EPAD - E, dtype=jnp.int32) % (NPAD - N)) + N
    src3 = jnp.concatenate([ei[0], pad]).reshape(NW, CPT, CHUNK)
    dst3 = jnp.concatenate([ei[1], pad]).reshape(NW, CPT, CHUNK)

    deg_parts = _deg_call(ei)
    g, dr = _mm_call(x, W, deg_parts)
    acc = _acc_call(src3, dst3, g)
    return _out_call(acc, dr)
